# Initial kernel scaffold; baseline (speedup 1.0000x reference)
import jax
import jax.numpy as jnp
from jax.experimental import pallas as pl


def kernel(inputs, table, gamma, beta):
    raise NotImplementedError("write your pallas kernel here")


# speedup vs baseline: 2.7939x; 2.7939x over previous
import functools
import math

import numpy as np
import jax
import jax.numpy as jnp
from jax import lax
from jax.experimental import pallas as pl
from jax.experimental.pallas import tpu as pltpu
from jax.experimental.pallas import tpu_sc as plsc

NUM_VOCAB = 100000
DIM = 128
PAD = 0
BATCH = 1024
SEQ_LEN = 200

NC = 2
NS = 16
NW = NC * NS
ROWS_PER_W = BATCH // NW
C0 = 104
C1 = 96
LPAD = C0 + C1 + 8
NG = LPAD // 16
NJ = DIM // 16
SCALE = math.sqrt(DIM)
EPS = 1e-5


def _positional_encoding(max_seq_len, dim):
    position = np.arange(0, max_seq_len, dtype=np.float64)[:, None]
    div_term = np.exp(-np.arange(0, dim, 2, dtype=np.float64) / dim
                      * math.log(10000.0))
    pe = np.zeros((max_seq_len, dim), dtype=np.float32)
    pe[:, 0::2] = np.sin(position * div_term).astype(np.float32)
    pe[:, 1::2] = np.cos(position * div_term).astype(np.float32)
    return pe


def _pe_padded():
    pe = _positional_encoding(SEQ_LEN, DIM)
    pad = np.zeros((LPAD, DIM), dtype=np.float32)
    pad[0:SEQ_LEN] = pe
    return pad


_PE_PAD = _pe_padded()


def _splat_f32(v):
    return jnp.full((16,), v, dtype=jnp.float32)


def _splat_i32(v):
    return jnp.full((16,), v, dtype=jnp.int32)


_GATHER_DN = lax.GatherDimensionNumbers(
    offset_dims=(), collapsed_slice_dims=(0,), start_index_map=(0,))


def _bcast_lane(v, k):
    idx = jnp.full((16, 1), k, dtype=jnp.int32)
    return lax.gather(v, idx, _GATHER_DN, slice_sizes=(1,),
                      mode=lax.GatherScatterMode.PROMISE_IN_BOUNDS)


def _rsqrt16(v):
    i = plsc.bitcast(v, jnp.int32)
    y = plsc.bitcast(_splat_i32(0x5F3759DF) - (i >> 1), jnp.float32)
    half_v = v * _splat_f32(0.5)
    three_half = _splat_f32(1.5)
    for _ in range(3):
        y = y * (three_half - half_v * y * y)
    return y


def _sc_kernel(idx_hbm, table_hbm, pe_hbm, gamma_hbm, beta_hbm, out_hbm,
               idx_v, rows_v, pe_v, gamma_v, beta_v, sem):
    wid = lax.axis_index("s") * NC + lax.axis_index("c")

    pltpu.sync_copy(pe_hbm, pe_v)
    pltpu.sync_copy(gamma_hbm, gamma_v)
    pltpu.sync_copy(beta_hbm, beta_v)

    g = [gamma_v[pl.ds(j * 16, 16)] for j in range(NJ)]
    bt = [beta_v[pl.ds(j * 16, 16)] for j in range(NJ)]
    scale_v = _splat_f32(SCALE)
    zero_v = _splat_f32(0.0)
    pad_v = _splat_i32(PAD)
    inv_d = _splat_f32(1.0 / DIM)
    eps_v = _splat_f32(EPS)

    def row_body(r, carry):
        row = wid * ROWS_PER_W + r
        pltpu.sync_copy(idx_hbm.at[row], idx_v)
        cp0 = pltpu.async_copy(table_hbm.at[idx_v.at[pl.ds(0, C0)]],
                               rows_v.at[pl.ds(0, C0)], sem)
        cp1 = pltpu.async_copy(table_hbm.at[idx_v.at[pl.ds(C0, C1)]],
                               rows_v.at[pl.ds(C0, C1)], sem)
        cp0.wait()
        cp1.wait()

        def grp_body(gi, c2):
            base = gi * 16
            ids = idx_v[pl.ds(base, 16)]
            for k in range(16):
                t = base + k
                tid = _bcast_lane(ids, k)
                fac = jnp.where(tid != pad_v, scale_v, zero_v)

                xs = []
                for j in range(NJ):
                    e = rows_v[t, pl.ds(j * 16, 16)]
                    p = pe_v[t, pl.ds(j * 16, 16)]
                    xs.append(e * fac + p)

                s = (xs[0] + xs[1]) + (xs[2] + xs[3]) \
                    + ((xs[4] + xs[5]) + (xs[6] + xs[7]))
                sq = [x * x for x in xs]
                s2 = (sq[0] + sq[1]) + (sq[2] + sq[3]) \
                    + ((sq[4] + sq[5]) + (sq[6] + sq[7]))

                mean_v = _splat_f32(jnp.sum(s)) * inv_d
                msq_v = _splat_f32(jnp.sum(s2)) * inv_d
                var_v = msq_v - mean_v * mean_v
                rstd = _rsqrt16(var_v + eps_v)

                for j in range(NJ):
                    out = (xs[j] - mean_v) * (rstd * g[j]) + bt[j]
                    rows_v[t, pl.ds(j * 16, 16)] = out
            return c2

        lax.fori_loop(0, NG, grp_body, 0)
        pltpu.sync_copy(rows_v.at[pl.ds(0, SEQ_LEN)], out_hbm.at[row])
        return carry

    lax.fori_loop(0, ROWS_PER_W, row_body, 0)


@jax.jit
def _run(idx, table, pe, gamma, beta):
    mesh = plsc.VectorSubcoreMesh(core_axis_name="c", subcore_axis_name="s")
    fn = functools.partial(
        pl.kernel,
        mesh=mesh,
        compiler_params=pltpu.CompilerParams(needs_layout_passes=False),
        out_type=jax.ShapeDtypeStruct((BATCH, SEQ_LEN, DIM), jnp.float32),
        scratch_types=[
            pltpu.VMEM((LPAD,), jnp.int32),
            pltpu.VMEM((LPAD, DIM), jnp.float32),
            pltpu.VMEM((LPAD, DIM), jnp.float32),
            pltpu.VMEM((DIM,), jnp.float32),
            pltpu.VMEM((DIM,), jnp.float32),
            pltpu.SemaphoreType.DMA,
        ],
    )(_sc_kernel)
    return fn(idx, table, pe, gamma, beta)


def kernel(inputs, table, gamma, beta):
    idx = jnp.pad(inputs.astype(jnp.int32), ((0, 0), (0, LPAD - SEQ_LEN)))
    pe = jnp.asarray(_PE_PAD)
    return _run(idx, table, pe, gamma, beta)
